# baseline (speedup 1.0000x reference)
import functools

import jax
import jax.numpy as jnp
from jax import lax
from jax.experimental import pallas as pl
from jax.experimental.pallas import tpu as pltpu
from jax.experimental.pallas import tpu_sc as plsc

B = 4096
L = 50
D = 64
NBINS = 32
LANES = 16
NC = 2
NS = 16
NW = NC * NS
ROWS_PER_W = B // NW
GROUPS = ROWS_PER_W // LANES
IDS_WORDS = ROWS_PER_W * L
COUNT_WORDS = ROWS_PER_W * NBINS

_mesh = plsc.VectorSubcoreMesh(core_axis_name="c", subcore_axis_name="s")


@functools.partial(
    pl.kernel,
    mesh=_mesh,
    out_type=jax.ShapeDtypeStruct((B * NBINS,), jnp.float32),
    scratch_types=[
        pltpu.VMEM((IDS_WORDS,), jnp.int32),
        pltpu.VMEM((IDS_WORDS,), jnp.int32),
        pltpu.VMEM((COUNT_WORDS,), jnp.float32),
        pltpu.SemaphoreType.DMA,
        pltpu.SemaphoreType.DMA,
    ],
    compiler_params=pltpu.CompilerParams(needs_layout_passes=False),
)
def _hist_kernel(ids_hbm, typ_hbm, counts_hbm, ids_v, typ_v, counts_v,
                 sem_i, sem_t):
    wid = lax.axis_index("s") * NC + lax.axis_index("c")
    base = wid * IDS_WORDS
    cp_i = pltpu.async_copy(ids_hbm.at[pl.ds(base, IDS_WORDS)], ids_v, sem_i)
    cp_t = pltpu.async_copy(typ_hbm.at[pl.ds(base, IDS_WORDS)], typ_v, sem_t)

    zeros = jnp.zeros((LANES,), jnp.float32)

    def zero_body(i, carry):
        b = i * (LANES * 8)
        for u in range(8):
            counts_v[pl.ds(b + u * LANES, LANES)] = zeros
        return carry

    lax.fori_loop(0, COUNT_WORDS // (LANES * 8), zero_body, 0)
    cp_i.wait()
    cp_t.wait()

    ones = jnp.ones((LANES,), jnp.float32)
    stride_vec = lax.iota(jnp.int32, LANES) * L
    lane_rows = lax.iota(jnp.int32, LANES) * NBINS
    UNROLL = 5
    for g in range(GROUPS):
        gvec = stride_vec + g * LANES * L
        row_vec = lane_rows + g * LANES * NBINS
        row_vec_t = row_vec + (NBINS - 5)

        def hist_body(j, carry, gvec=gvec, row_vec=row_vec, row_vec_t=row_vec_t):
            idx = gvec + j * UNROLL
            for u in range(UNROLL):
                c = plsc.load_gather(ids_v, [idx + u])
                plsc.addupdate_scatter(counts_v, [c + row_vec], ones)
                t = plsc.load_gather(typ_v, [idx + u])
                plsc.addupdate_scatter(counts_v, [t + row_vec_t], ones)
            return carry

        lax.fori_loop(0, L // UNROLL, hist_body, 0)

    pltpu.sync_copy(counts_v, counts_hbm.at[pl.ds(wid * COUNT_WORDS, COUNT_WORDS)])


def _matmul_body(counts_ref, table_ref, out_ref):
    out_ref[...] = jnp.dot(
        counts_ref[...], table_ref[...], preferred_element_type=jnp.float32
    )


_ROW_BLK = 512


def _pooled_matmul(counts, table32):
    return pl.pallas_call(
        _matmul_body,
        out_shape=jax.ShapeDtypeStruct((B, D), jnp.float32),
    )(counts, table32)


@jax.jit
def kernel(marker_ids, marker_types, marker_embed, marker_type_embed):
    ids = marker_ids.astype(jnp.int32).reshape(-1)
    typ = marker_types.astype(jnp.int32).reshape(-1)

    counts = _hist_kernel(ids, typ).reshape(B, NBINS)

    table32 = jnp.concatenate(
        [
            marker_embed,
            marker_type_embed,
            jnp.zeros((NBINS - marker_embed.shape[0] - marker_type_embed.shape[0], D),
                      jnp.float32),
        ],
        axis=0,
    ) * (1.0 / L)

    return _pooled_matmul(counts, table32)
